# baseline (speedup 1.0000x reference)
import functools
import math

import jax
import jax.numpy as jnp
from jax import lax
from jax.experimental import pallas as pl
from jax.experimental.pallas import tpu as pltpu
from jax.experimental.pallas import tpu_sc as plsc

D_MODEL = 1024
SCALE = math.sqrt(D_MODEL)
LANES = 16
NUM_CORES = 2
NUM_SUBCORES = 16
NUM_WORKERS = NUM_CORES * NUM_SUBCORES
CHUNK = 16
NBUF = 2


@jax.jit
def _embed(idx, table):
    B = idx.shape[0]
    b_per_w = B // NUM_WORKERS
    nchunks = b_per_w // CHUNK

    mesh = plsc.VectorSubcoreMesh(core_axis_name="c", subcore_axis_name="s")

    @functools.partial(
        pl.kernel,
        out_type=jax.ShapeDtypeStruct((B, D_MODEL), jnp.float32),
        mesh=mesh,
        scratch_types=[
            pltpu.VMEM((b_per_w,), jnp.int32),
            pltpu.VMEM((NBUF, CHUNK, D_MODEL), jnp.float32),
            pltpu.VMEM((NBUF, CHUNK, D_MODEL), jnp.float32),
            pltpu.SemaphoreType.DMA((NBUF,)),
            pltpu.SemaphoreType.DMA((NBUF,)),
        ],
    )
    def body(table_hbm, idx_hbm, out_hbm, idx_v, gbuf, obuf, gsem, osem):
        wid = lax.axis_index("s") * NUM_CORES + lax.axis_index("c")
        base = wid * b_per_w
        pltpu.sync_copy(idx_hbm.at[pl.ds(base, b_per_w)], idx_v)

        def start_gather(c, b):
            pltpu.make_async_copy(
                table_hbm.at[idx_v.at[pl.ds(c * CHUNK, CHUNK)]],
                gbuf.at[b],
                gsem.at[b],
            ).start()

        def wait_gather(b):
            pltpu.make_async_copy(
                table_hbm.at[idx_v.at[pl.ds(0, CHUNK)]],
                gbuf.at[b],
                gsem.at[b],
            ).wait()

        def start_out(c, b):
            pltpu.make_async_copy(
                obuf.at[b],
                out_hbm.at[pl.ds(base + c * CHUNK, CHUNK)],
                osem.at[b],
            ).start()

        def wait_out(b):
            pltpu.make_async_copy(
                obuf.at[b],
                out_hbm.at[pl.ds(base, CHUNK)],
                osem.at[b],
            ).wait()

        for b in range(NBUF):
            start_gather(b, b)

        def step(i, carry):
            for b in range(NBUF):
                c = i * NBUF + b
                wait_gather(b)

                @pl.when(c >= NBUF)
                def _():
                    wait_out(b)

                def scale_row(r, acc):
                    for j in range(D_MODEL // LANES):
                        sl = pl.ds(j * LANES, LANES)
                        obuf[b, r, sl] = gbuf[b, r, sl] * SCALE
                    return acc

                lax.fori_loop(0, CHUNK, scale_row, 0)
                start_out(c, b)

                @pl.when(c + NBUF < nchunks)
                def _():
                    start_gather(c + NBUF, b)

            return carry

        lax.fori_loop(0, nchunks // NBUF, step, 0)
        for b in range(NBUF):
            wait_out(b)

    return body(table, idx)


def kernel(x, table):
    idx = x.reshape(-1).astype(jnp.int32)
    out = _embed(idx, table)
    return out.reshape(x.shape + (D_MODEL,))


# speedup vs baseline: 1.4863x; 1.4863x over previous
import functools
import math

import jax
import jax.numpy as jnp
from jax import lax
from jax.experimental import pallas as pl
from jax.experimental.pallas import tpu as pltpu
from jax.experimental.pallas import tpu_sc as plsc

D_MODEL = 1024
SCALE = math.sqrt(D_MODEL)
LANES = 16
NUM_CORES = 2
NUM_SUBCORES = 16
NUM_WORKERS = NUM_CORES * NUM_SUBCORES
CHUNK = 32
NBUF = 3


@jax.jit
def _embed(idx, table):
    B = idx.shape[0]
    b_per_w = B // NUM_WORKERS
    nchunks = b_per_w // CHUNK

    mesh = plsc.VectorSubcoreMesh(core_axis_name="c", subcore_axis_name="s")

    @functools.partial(
        pl.kernel,
        out_type=jax.ShapeDtypeStruct((B, D_MODEL), jnp.float32),
        mesh=mesh,
        scratch_types=[
            pltpu.VMEM((b_per_w,), jnp.int32),
            pltpu.VMEM((NBUF, CHUNK, D_MODEL), jnp.float32),
            pltpu.SemaphoreType.DMA((NBUF,)),
            pltpu.SemaphoreType.DMA((NBUF,)),
        ],
    )
    def body(table_hbm, idx_hbm, out_hbm, idx_v, rows_v, gsem, osem):
        wid = lax.axis_index("s") * NUM_CORES + lax.axis_index("c")
        base = wid * b_per_w
        pltpu.sync_copy(idx_hbm.at[pl.ds(base, b_per_w)], idx_v)

        def start_gather(c, b):
            pltpu.make_async_copy(
                table_hbm.at[idx_v.at[pl.ds(c * CHUNK, CHUNK)]],
                rows_v.at[b],
                gsem.at[b],
            ).start()

        def wait_gather(b):
            pltpu.make_async_copy(
                table_hbm.at[idx_v.at[pl.ds(0, CHUNK)]],
                rows_v.at[b],
                gsem.at[b],
            ).wait()

        def start_out(c, b):
            pltpu.make_async_copy(
                rows_v.at[b],
                out_hbm.at[pl.ds(base + c * CHUNK, CHUNK)],
                osem.at[b],
            ).start()

        def wait_out(b):
            pltpu.make_async_copy(
                rows_v.at[b],
                out_hbm.at[pl.ds(base, CHUNK)],
                osem.at[b],
            ).wait()

        start_gather(0, 0)
        start_gather(1, 1)

        def step(c, carry):
            b = lax.rem(c, NBUF)
            wait_gather(b)

            def scale_row(r, acc):
                for j in range(D_MODEL // LANES):
                    sl = pl.ds(j * LANES, LANES)
                    rows_v[b, r, sl] = rows_v[b, r, sl] * SCALE
                return acc

            lax.fori_loop(0, CHUNK, scale_row, 0)
            start_out(c, b)

            b2 = lax.rem(c + 2, NBUF)

            @pl.when(c + 2 < nchunks)
            def _():
                @pl.when(c >= 1)
                def _():
                    wait_out(b2)

                start_gather(c + 2, b2)

            return carry

        lax.fori_loop(0, nchunks, step, 0)
        for b in range(NBUF):
            wait_out(b)

    return body(table, idx)


def kernel(x, table):
    idx = x.reshape(-1).astype(jnp.int32)
    out = _embed(idx, table)
    return out.reshape(x.shape + (D_MODEL,))
